# Initial kernel scaffold; baseline (speedup 1.0000x reference)
import jax
import jax.numpy as jnp
from jax.experimental import pallas as pl


def kernel(x, edge_index, W):
    raise NotImplementedError("write your pallas kernel here")


# speedup vs baseline: 110.9880x; 110.9880x over previous
import functools

import jax
import jax.numpy as jnp
from jax import lax
from jax.experimental import pallas as pl
from jax.experimental.pallas import tpu as pltpu
from jax.experimental.pallas import tpu_sc as plsc

N = 10000
E = 320000
D = 128
NP = 10240
NB = NP // 128
NC = 2
NS = 16
NW = NC * NS
EPW = E // NW
VPW = EPW // 16

_mesh = plsc.VectorSubcoreMesh(core_axis_name="c", subcore_axis_name="s")


@functools.partial(
    pl.kernel,
    mesh=_mesh,
    out_type=jax.ShapeDtypeStruct((NW, NP), jnp.float32),
    scratch_types=[
        pltpu.VMEM((EPW,), jnp.int32),
        pltpu.VMEM((NP,), jnp.float32),
    ],
    compiler_params=pltpu.CompilerParams(needs_layout_passes=False),
)
def _sc_degree(dst_hbm, out_hbm, dst_v, acc_v):
    wid = lax.axis_index("s") * NC + lax.axis_index("c")
    pltpu.sync_copy(dst_hbm.at[pl.ds(wid * EPW, EPW)], dst_v)

    zeros = jnp.zeros((16,), jnp.float32)

    def zbody(i, carry):
        acc_v[pl.ds(i * 16, 16)] = zeros
        return carry

    lax.fori_loop(0, NP // 16, zbody, 0)

    ones = jnp.ones((16,), jnp.float32)

    def body(i, carry):
        idx = dst_v[pl.ds(i * 16, 16)]
        plsc.addupdate_scatter(acc_v, [idx], ones)
        return carry

    lax.fori_loop(0, VPW, body, 0)
    pltpu.sync_copy(acc_v, out_hbm.at[wid])


@functools.partial(
    pl.kernel,
    mesh=_mesh,
    out_type=jax.ShapeDtypeStruct((NW, NP), jnp.float32),
    scratch_types=[
        pltpu.VMEM((EPW,), jnp.int32),
        pltpu.VMEM((EPW,), jnp.int32),
        pltpu.VMEM((NP,), jnp.float32),
        pltpu.VMEM((NP,), jnp.float32),
    ],
    compiler_params=pltpu.CompilerParams(needs_layout_passes=False),
)
def _sc_aggregate(src_hbm, dst_hbm, v_hbm, out_hbm, src_v, dst_v, vv, acc_v):
    wid = lax.axis_index("s") * NC + lax.axis_index("c")
    pltpu.sync_copy(src_hbm.at[pl.ds(wid * EPW, EPW)], src_v)
    pltpu.sync_copy(dst_hbm.at[pl.ds(wid * EPW, EPW)], dst_v)
    pltpu.sync_copy(v_hbm, vv)

    zeros = jnp.zeros((16,), jnp.float32)

    def zbody(i, carry):
        acc_v[pl.ds(i * 16, 16)] = zeros
        return carry

    lax.fori_loop(0, NP // 16, zbody, 0)

    def body(i, carry):
        s16 = src_v[pl.ds(i * 16, 16)]
        d16 = dst_v[pl.ds(i * 16, 16)]
        vals = plsc.load_gather(vv, [s16])
        plsc.addupdate_scatter(acc_v, [d16], vals)
        return carry

    lax.fori_loop(0, VPW, body, 0)
    pltpu.sync_copy(acc_v, out_hbm.at[wid])


def _tc_prep_body(x3_ref, w_ref, cnt_ref, v_ref, dinv_ref):
    w3 = w_ref[...].reshape(1, 1, D)
    xw = jnp.sum(x3_ref[...] * w3, axis=2)
    deg = jnp.sum(cnt_ref[...], axis=0) + 1.0
    dinv = lax.rsqrt(deg)
    dinv_ref[...] = dinv
    v_ref[...] = xw * dinv


_tc_prep = pl.pallas_call(
    _tc_prep_body,
    out_shape=[
        jax.ShapeDtypeStruct((NB, 128), jnp.float32),
        jax.ShapeDtypeStruct((NB, 128), jnp.float32),
    ],
)


def _tc_finish_body(p_ref, v_ref, dinv_ref, o_ref):
    s = jnp.sum(p_ref[...], axis=0) + v_ref[...]
    o_ref[...] = dinv_ref[...] * s


_tc_finish = pl.pallas_call(
    _tc_finish_body,
    out_shape=jax.ShapeDtypeStruct((NB, 128), jnp.float32),
)


def kernel(x, edge_index, W):
    src = edge_index[0]
    dst = edge_index[1]
    x3 = jnp.pad(x, ((0, NP - N), (0, 0))).reshape(NB, 128, D)
    w_row = W.reshape(1, D)

    counts = _sc_degree(dst)
    v2d, dinv2d = _tc_prep(x3, w_row, counts.reshape(NW, NB, 128))
    parts = _sc_aggregate(src, dst, v2d.reshape(NP))
    out2d = _tc_finish(parts.reshape(NW, NB, 128), v2d, dinv2d)
    return out2d.reshape(NP)[:N]


# speedup vs baseline: 122.1596x; 1.1007x over previous
import functools

import jax
import jax.numpy as jnp
from jax import lax
from jax.experimental import pallas as pl
from jax.experimental.pallas import tpu as pltpu
from jax.experimental.pallas import tpu_sc as plsc

N = 10000
E = 320000
D = 128
NP = 10240
NB = NP // 128
NC = 2
NS = 16
NW = NC * NS
EPW = E // NW
VPW = EPW // 16
LU = 5
ZU = 8

_mesh = plsc.VectorSubcoreMesh(core_axis_name="c", subcore_axis_name="s")


@functools.partial(
    pl.kernel,
    mesh=_mesh,
    out_type=jax.ShapeDtypeStruct((NW, NP), jnp.float32),
    scratch_types=[
        pltpu.VMEM((EPW,), jnp.int32),
        pltpu.VMEM((NP,), jnp.float32),
        pltpu.SemaphoreType.DMA,
    ],
    compiler_params=pltpu.CompilerParams(needs_layout_passes=False),
)
def _sc_degree(dst_hbm, out_hbm, dst_v, acc_v, sem):
    wid = lax.axis_index("s") * NC + lax.axis_index("c")
    cp = pltpu.async_copy(dst_hbm.at[pl.ds(wid * EPW, EPW)], dst_v, sem)

    zeros = jnp.zeros((16,), jnp.float32)

    def zbody(i, carry):
        base = i * (16 * ZU)
        for k in range(ZU):
            acc_v[pl.ds(base + k * 16, 16)] = zeros
        return carry

    lax.fori_loop(0, NP // (16 * ZU), zbody, 0)
    cp.wait()

    ones = jnp.ones((16,), jnp.float32)

    def body(i, carry):
        base = i * (16 * LU)
        for k in range(LU):
            idx = dst_v[pl.ds(base + k * 16, 16)]
            plsc.addupdate_scatter(acc_v, [idx], ones)
        return carry

    lax.fori_loop(0, VPW // LU, body, 0)
    pltpu.sync_copy(acc_v, out_hbm.at[wid])


@functools.partial(
    pl.kernel,
    mesh=_mesh,
    out_type=jax.ShapeDtypeStruct((NW, NP), jnp.float32),
    scratch_types=[
        pltpu.VMEM((EPW,), jnp.int32),
        pltpu.VMEM((EPW,), jnp.int32),
        pltpu.VMEM((NP,), jnp.float32),
        pltpu.VMEM((NP,), jnp.float32),
        pltpu.SemaphoreType.DMA,
    ],
    compiler_params=pltpu.CompilerParams(needs_layout_passes=False),
)
def _sc_aggregate(src_hbm, dst_hbm, v_hbm, out_hbm, src_v, dst_v, vv, acc_v, sem):
    wid = lax.axis_index("s") * NC + lax.axis_index("c")
    c1 = pltpu.async_copy(src_hbm.at[pl.ds(wid * EPW, EPW)], src_v, sem)
    c2 = pltpu.async_copy(dst_hbm.at[pl.ds(wid * EPW, EPW)], dst_v, sem)
    c3 = pltpu.async_copy(v_hbm, vv, sem)

    zeros = jnp.zeros((16,), jnp.float32)

    def zbody(i, carry):
        base = i * (16 * ZU)
        for k in range(ZU):
            acc_v[pl.ds(base + k * 16, 16)] = zeros
        return carry

    lax.fori_loop(0, NP // (16 * ZU), zbody, 0)
    c1.wait()
    c2.wait()
    c3.wait()

    def body(i, carry):
        base = i * (16 * LU)
        for k in range(LU):
            s16 = src_v[pl.ds(base + k * 16, 16)]
            d16 = dst_v[pl.ds(base + k * 16, 16)]
            vals = plsc.load_gather(vv, [s16])
            plsc.addupdate_scatter(acc_v, [d16], vals)
        return carry

    lax.fori_loop(0, VPW // LU, body, 0)
    pltpu.sync_copy(acc_v, out_hbm.at[wid])


def _tc_prep_body(x3_ref, w_ref, cnt_ref, v_ref, dinv_ref):
    w3 = w_ref[...].reshape(1, 1, D)
    xw = jnp.sum(x3_ref[...] * w3, axis=2)
    deg = jnp.sum(cnt_ref[...], axis=0) + 1.0
    dinv = lax.rsqrt(deg)
    dinv_ref[...] = dinv
    v_ref[...] = xw * dinv


_tc_prep = pl.pallas_call(
    _tc_prep_body,
    out_shape=[
        jax.ShapeDtypeStruct((NB, 128), jnp.float32),
        jax.ShapeDtypeStruct((NB, 128), jnp.float32),
    ],
)


def _tc_finish_body(p_ref, v_ref, dinv_ref, o_ref):
    s = jnp.sum(p_ref[...], axis=0) + v_ref[...]
    o_ref[...] = dinv_ref[...] * s


_tc_finish = pl.pallas_call(
    _tc_finish_body,
    out_shape=jax.ShapeDtypeStruct((NB, 128), jnp.float32),
)


def kernel(x, edge_index, W):
    src = edge_index[0]
    dst = edge_index[1]
    x3 = jnp.pad(x, ((0, NP - N), (0, 0))).reshape(NB, 128, D)
    w_row = W.reshape(1, D)

    counts = _sc_degree(dst)
    v2d, dinv2d = _tc_prep(x3, w_row, counts.reshape(NW, NB, 128))
    parts = _sc_aggregate(src, dst, v2d.reshape(NP))
    out2d = _tc_finish(parts.reshape(NW, NB, 128), v2d, dinv2d)
    return out2d.reshape(NP)[:N]
